# baseline (speedup 1.0000x reference)
import jax
import jax.numpy as jnp
from jax import lax
from jax.experimental import pallas as pl
from jax.experimental.pallas import tpu as pltpu
from jax.experimental.pallas import tpu_sc as plsc

_NUM_CORES = 2
_NUM_SUBCORES = 16
_NBUF = 4


def kernel(x, table):
    batch, seq = x.shape
    vocab, emb = table.shape
    nw = _NUM_CORES * _NUM_SUBCORES
    bpw = batch // nw
    nch = bpw

    seq_pad = 64
    idx = jnp.pad(x.astype(jnp.int32), ((0, 0), (0, seq_pad - seq))).reshape(-1)
    mesh = plsc.VectorSubcoreMesh(core_axis_name="c", subcore_axis_name="s")

    @pl.kernel(
        out_type=jax.ShapeDtypeStruct((batch, seq, emb), jnp.float32),
        mesh=mesh,
        compiler_params=pltpu.CompilerParams(use_tc_tiling_on_sc=False),
        scratch_types=[
            pltpu.VMEM((bpw * seq_pad,), jnp.int32),
            pltpu.VMEM((_NBUF, seq, emb), jnp.float32),
            pltpu.SemaphoreType.DMA((_NBUF,)),
            pltpu.SemaphoreType.DMA((_NBUF,)),
        ],
    )
    def k(table_hbm, i_hbm, o_hbm, idx_v, buf, gsem, osem):
        wid = lax.axis_index("s") * _NUM_CORES + lax.axis_index("c")
        base = wid * bpw
        pltpu.sync_copy(i_hbm.at[pl.ds(base * seq_pad, bpw * seq_pad)], idx_v)

        def g_copy(g, b):
            return pltpu.make_async_copy(
                table_hbm.at[idx_v.at[pl.ds(g * seq_pad, seq)]],
                buf.at[b],
                gsem.at[b])

        def o_copy(g, b):
            return pltpu.make_async_copy(
                buf.at[b], o_hbm.at[base + g], osem.at[b])

        for b in range(_NBUF):
            g_copy(b, b).start()
        for g in (0, 1, 2, 3):
            b = g % _NBUF
            g_copy(g, b).wait()
            o_copy(g, b).start()
            if g >= 1:
                o_copy(g - 1, (g - 1) % _NBUF).wait()
                g_copy(g + 3, (g - 1) % _NBUF).start()

        @pl.loop(1, nch // _NBUF - 1)
        def _(c):
            for b in range(_NBUF):
                g = c * _NBUF + b
                g_copy(g, b).wait()
                o_copy(g, b).start()
                o_copy(g - 1, (g - 1) % _NBUF).wait()
                g_copy(g + 3, (g + 3) % _NBUF).start()

        for g in range(nch - _NBUF, nch):
            b = g % _NBUF
            g_copy(g, b).wait()
            o_copy(g, b).start()
            o_copy(g - 1, (g - 1) % _NBUF).wait()
            if g == nch - _NBUF:
                g_copy(nch - 1, (nch - 1) % _NBUF).start()
        o_copy(nch - 1, (nch - 1) % _NBUF).wait()

    return k(table, idx)
